# Initial kernel scaffold; baseline (speedup 1.0000x reference)
import jax
import jax.numpy as jnp
from jax.experimental import pallas as pl


def kernel(input):
    raise NotImplementedError("write your pallas kernel here")


# speedup vs baseline: 9.6930x; 9.6930x over previous
import jax
import jax.numpy as jnp
from jax import lax
from jax.experimental import pallas as pl
from jax.experimental.pallas import tpu as pltpu
from jax.experimental.pallas import tpu_sc as plsc

R, N = 128, 32768
L = 16
NCH = N // L
NC, NS = 2, 16
NW = NC * NS
RPW = R // NW
NEWTON_ITERS = 26


def _sparsemax_body(x_hbm, out_hbm, row_v, cand_v, cidx_v):
    wid = lax.axis_index("c") * NS + lax.axis_index("s")
    lanes = lax.broadcasted_iota(jnp.int32, (L,), 0)
    ones_i = jnp.ones((L,), jnp.int32)
    zeros_i = jnp.zeros((L,), jnp.int32)
    zeros_f = jnp.zeros((L,), jnp.float32)

    def do_row(i, _):
        r = wid * RPW + i
        pltpu.sync_copy(x_hbm.at[r], row_v)

        def maxbody(j, acc):
            return jnp.maximum(acc, row_v[pl.ds(j * L, L)])

        acc = lax.fori_loop(0, NCH, maxbody,
                            jnp.full((L,), -jnp.inf, jnp.float32))
        rowmax = jnp.max(acc)
        tau0 = rowmax - 1.0

        def cbody(j, off):
            v = row_v[pl.ds(j * L, L)]
            m = v > tau0

            def compact(o):
                plsc.store_compressed(cand_v.at[pl.ds(o, L)], v, mask=m)
                plsc.store_compressed(cidx_v.at[pl.ds(o, L)], j * L + lanes,
                                      mask=m)
                return o + jnp.sum(jnp.where(m, ones_i, zeros_i))

            off = lax.cond(jnp.any(m), compact, lambda o: o, off)
            row_v[pl.ds(j * L, L)] = zeros_f
            return off

        off = lax.fori_loop(0, NCH, cbody, jnp.int32(0))
        cand_v[pl.ds(off, L)] = jnp.full((L,), tau0 - 1.0, jnp.float32)
        ncand_ch = lax.shift_right_logical(off + (L - 1), 4)

        ones_f = jnp.ones((L,), jnp.float32)

        def evalf(t_v):
            def ebody(j, carry):
                sv, cv = carry
                v = cand_v[pl.ds(j * L, L)]
                m = v > t_v
                return sv + jnp.where(m, v, zeros_f), \
                    cv + jnp.where(m, ones_f, zeros_f)

            sv, cv = lax.fori_loop(0, ncand_ch, ebody, (zeros_f, zeros_f))
            return (jnp.broadcast_to(jnp.sum(sv), (L,)),
                    jnp.broadcast_to(jnp.sum(cv), (L,)))

        tau0_v = jnp.broadcast_to(tau0, (L,))
        rowmax_v = jnp.broadcast_to(rowmax, (L,))
        s0_v, c0_v = evalf(tau0_v)

        def nbody(_, carry):
            lo, hi, s_lo, c_lo = carry
            t_newton = (s_lo - ones_f) / c_lo
            t = jnp.minimum(jnp.maximum(t_newton, 0.5 * (lo + hi)), hi)
            s_t, c_t = evalf(t)
            f_t = s_t - t * c_t - ones_f
            good = f_t >= zeros_f
            return (jnp.where(good, t, lo), jnp.where(good, hi, t),
                    jnp.where(good, s_t, s_lo), jnp.where(good, c_t, c_lo))

        lo, hi, s_lo, c_lo = lax.fori_loop(
            0, NEWTON_ITERS, nbody, (tau0_v, rowmax_v, s0_v, c0_v))
        tau = (s_lo - ones_f) / c_lo

        def sbody(j, _):
            v = cand_v[pl.ds(j * L, L)]
            idxv = cidx_v[pl.ds(j * L, L)]
            m = (j * L + lanes) < off
            plsc.store_scatter(row_v, [idxv], jnp.maximum(v - tau, 0.0),
                               mask=m)
            return _

        lax.fori_loop(0, ncand_ch, sbody, jnp.int32(0))
        pltpu.sync_copy(row_v, out_hbm.at[r])
        return _

    lax.fori_loop(0, RPW, do_row, jnp.int32(0))


def kernel(input):
    mesh = plsc.VectorSubcoreMesh(core_axis_name="c", subcore_axis_name="s",
                                  num_cores=NC, num_subcores=NS)
    f = pl.kernel(
        _sparsemax_body,
        out_type=jax.ShapeDtypeStruct((R, N), jnp.float32),
        mesh=mesh,
        compiler_params=pltpu.CompilerParams(needs_layout_passes=False),
        scratch_types=[
            pltpu.VMEM((N,), jnp.float32),
            pltpu.VMEM((N + L,), jnp.float32),
            pltpu.VMEM((N + L,), jnp.int32),
        ],
    )
    return f(input)


# speedup vs baseline: 30.3146x; 3.1275x over previous
import jax
import jax.numpy as jnp
from jax import lax
from jax.experimental import pallas as pl
from jax.experimental.pallas import tpu as pltpu
from jax.experimental.pallas import tpu_sc as plsc

R, N = 128, 32768
L = 16
NCH = N // L
NC, NS = 2, 16
NW = NC * NS
RPW = R // NW
NEWTON_ITERS = 26


def _sparsemax_body(x_hbm, out_hbm, row_v, cand_v, cidx_v):
    wid = lax.axis_index("c") * NS + lax.axis_index("s")
    lanes = lax.broadcasted_iota(jnp.int32, (L,), 0)
    ones_i = jnp.ones((L,), jnp.int32)
    zeros_i = jnp.zeros((L,), jnp.int32)
    zeros_f = jnp.zeros((L,), jnp.float32)

    U = 8
    NG = NCH // U

    def do_row(i, _):
        r = wid * RPW + i
        pltpu.sync_copy(x_hbm.at[r], row_v)

        def maxbody(g, accs):
            base = g * (U * L)
            return tuple(
                jnp.maximum(accs[k], row_v[pl.ds(base + k * L, L)])
                for k in range(U))

        accs = lax.fori_loop(
            0, NG, maxbody,
            tuple(jnp.full((L,), -jnp.inf, jnp.float32) for _ in range(U)))
        acc = accs[0]
        for k in range(1, U):
            acc = jnp.maximum(acc, accs[k])
        rowmax = jnp.max(acc)
        tau0 = rowmax - 1.0

        def cbody(g, off):
            base = g * (U * L)
            vs = [row_v[pl.ds(base + k * L, L)] for k in range(U)]
            ms = [v > tau0 for v in vs]
            anym = ms[0]
            for k in range(1, U):
                anym = jnp.logical_or(anym, ms[k])

            def compact(o):
                for k in range(U):
                    plsc.store_compressed(cand_v.at[pl.ds(o, L)], vs[k],
                                          mask=ms[k])
                    plsc.store_compressed(cidx_v.at[pl.ds(o, L)],
                                          base + k * L + lanes, mask=ms[k])
                    o = o + jnp.sum(jnp.where(ms[k], ones_i, zeros_i))
                return o

            off = lax.cond(jnp.any(anym), compact, lambda o: o, off)
            for k in range(U):
                row_v[pl.ds(base + k * L, L)] = zeros_f
            return off

        off = lax.fori_loop(0, NG, cbody, jnp.int32(0))
        cand_v[pl.ds(off, L)] = jnp.full((L,), tau0 - 1.0, jnp.float32)
        ncand_ch = lax.shift_right_logical(off + (L - 1), 4)

        ones_f = jnp.ones((L,), jnp.float32)

        def evalf(t_v):
            def ebody(j, carry):
                sv, cv = carry
                v = cand_v[pl.ds(j * L, L)]
                m = v > t_v
                return sv + jnp.where(m, v, zeros_f), \
                    cv + jnp.where(m, ones_f, zeros_f)

            sv, cv = lax.fori_loop(0, ncand_ch, ebody, (zeros_f, zeros_f))
            return (jnp.broadcast_to(jnp.sum(sv), (L,)),
                    jnp.broadcast_to(jnp.sum(cv), (L,)))

        tau0_v = jnp.broadcast_to(tau0, (L,))
        rowmax_v = jnp.broadcast_to(rowmax, (L,))
        s0_v, c0_v = evalf(tau0_v)

        def nbody(_, carry):
            lo, hi, s_lo, c_lo = carry
            t_newton = (s_lo - ones_f) / c_lo
            t = jnp.minimum(jnp.maximum(t_newton, 0.5 * (lo + hi)), hi)
            s_t, c_t = evalf(t)
            f_t = s_t - t * c_t - ones_f
            good = f_t >= zeros_f
            return (jnp.where(good, t, lo), jnp.where(good, hi, t),
                    jnp.where(good, s_t, s_lo), jnp.where(good, c_t, c_lo))

        lo, hi, s_lo, c_lo = lax.fori_loop(
            0, NEWTON_ITERS, nbody, (tau0_v, rowmax_v, s0_v, c0_v))
        tau = (s_lo - ones_f) / c_lo

        def sbody(j, _):
            v = cand_v[pl.ds(j * L, L)]
            idxv = cidx_v[pl.ds(j * L, L)]
            m = (j * L + lanes) < off
            plsc.store_scatter(row_v, [idxv], jnp.maximum(v - tau, 0.0),
                               mask=m)
            return _

        lax.fori_loop(0, ncand_ch, sbody, jnp.int32(0))
        pltpu.sync_copy(row_v, out_hbm.at[r])
        return _

    lax.fori_loop(0, RPW, do_row, jnp.int32(0))


def kernel(input):
    mesh = plsc.VectorSubcoreMesh(core_axis_name="c", subcore_axis_name="s",
                                  num_cores=NC, num_subcores=NS)
    f = pl.kernel(
        _sparsemax_body,
        out_type=jax.ShapeDtypeStruct((R, N), jnp.float32),
        mesh=mesh,
        compiler_params=pltpu.CompilerParams(needs_layout_passes=False),
        scratch_types=[
            pltpu.VMEM((N,), jnp.float32),
            pltpu.VMEM((N + L,), jnp.float32),
            pltpu.VMEM((N + L,), jnp.int32),
        ],
    )
    return f(input)


# speedup vs baseline: 30.8306x; 1.0170x over previous
import jax
import jax.numpy as jnp
from jax import lax
from jax.experimental import pallas as pl
from jax.experimental.pallas import tpu as pltpu
from jax.experimental.pallas import tpu_sc as plsc

R, N = 128, 32768
L = 16
NC, NS = 2, 16
NW = NC * NS
RPW = R // NW
KPG = 16
GSZ = KPG * L
NG = N // GSZ
NEWTON_ITERS = 26


def _sparsemax_body(x_hbm, out_hbm, in0_v, in1_v, out_v, gmax_v,
                    pl0_v, pl1_v, sem_in0, sem_in1, sem_out):
    wid = lax.axis_index("c") * NS + lax.axis_index("s")
    lanes = lax.broadcasted_iota(jnp.int32, (L,), 0)
    lanes16 = lanes * L
    ones_i = jnp.ones((L,), jnp.int32)
    zeros_i = jnp.zeros((L,), jnp.int32)
    zeros_f = jnp.zeros((L,), jnp.float32)
    ones_f = jnp.ones((L,), jnp.float32)

    row0 = wid * RPW
    in_bufs = [in0_v, in1_v]
    in_sems = [sem_in0, sem_in1]
    pls = [pl0_v, pl1_v]

    cp_in = [None, None]
    cp_in[0] = pltpu.async_copy(x_hbm.at[row0], in0_v, sem_in0)

    def zbody(g, _):
        base = lax.shift_left(g, 8)
        for k in range(KPG):
            out_v[pl.ds(base + k * L, L)] = zeros_f
        return _

    lax.fori_loop(0, NG, zbody, jnp.int32(0))

    cp_out = None
    prev = None

    for i in range(RPW):
        b = i & 1
        in_ref = in_bufs[b]
        pl_ref = pls[b]
        cp_in[b].wait()
        if i + 1 < RPW:
            cp_in[1 - b] = pltpu.async_copy(
                x_hbm.at[row0 + i + 1], in_bufs[1 - b], in_sems[1 - b])

        def maxbody(g, acc):
            base = lax.shift_left(g, 8)
            gm = in_ref[pl.ds(base, L)]
            for k in range(1, KPG):
                gm = jnp.maximum(gm, in_ref[pl.ds(base + k * L, L)])
            gmax_v[pl.ds(lax.shift_left(g, 4), L)] = gm
            return jnp.maximum(acc, gm)

        acc = lax.fori_loop(0, NG, maxbody,
                            jnp.full((L,), -jnp.inf, jnp.float32))
        rowmax = jnp.max(acc)
        tau0 = rowmax - 1.0

        def gscan(g, np_):
            gm = gmax_v[pl.ds(lax.shift_left(g, 4), L)]
            m = gm > tau0

            def compact(o):
                plsc.store_compressed(pl_ref.at[pl.ds(o, L)],
                                      lax.shift_left(g, 8) + lanes, mask=m)
                return o + jnp.sum(jnp.where(m, ones_i, zeros_i))

            return lax.cond(jnp.any(m), compact, lambda o: o, np_)

        np_ = lax.fori_loop(0, NG, gscan, jnp.int32(0))

        def evalf(t_v):
            def ebody(e, carry):
                sv, cv = carry
                pv = pl_ref[pl.ds(e, L)]
                idx = pv[0] + lanes16
                v = plsc.load_gather(in_ref, [idx])
                m = v > t_v
                return sv + jnp.where(m, v, zeros_f), \
                    cv + jnp.where(m, ones_f, zeros_f)

            sv, cv = lax.fori_loop(0, np_, ebody, (zeros_f, zeros_f))
            return (jnp.broadcast_to(jnp.sum(sv), (L,)),
                    jnp.broadcast_to(jnp.sum(cv), (L,)))

        tau0_v = jnp.broadcast_to(tau0, (L,))
        s0_v, c0_v = evalf(tau0_v)

        def ncond(carry):
            _, _, _, _, done, it = carry
            return jnp.logical_and(jnp.logical_not(done), it < NEWTON_ITERS)

        def nbody(carry):
            lo, hi, s_lo, c_lo, _, it = carry
            t_newton = (s_lo - ones_f) / c_lo
            done = jnp.all(t_newton <= lo)
            t = jnp.minimum(jnp.maximum(t_newton, 0.5 * (lo + hi)), hi)
            s_t, c_t = evalf(t)
            f_t = s_t - t * c_t - ones_f
            good = f_t >= zeros_f
            return (jnp.where(good, t, lo), jnp.where(good, hi, t),
                    jnp.where(good, s_t, s_lo), jnp.where(good, c_t, c_lo),
                    done, it + 1)

        lo, hi, s_lo, c_lo, _, _ = lax.while_loop(
            ncond, nbody,
            (tau0_v, jnp.broadcast_to(rowmax, (L,)), s0_v, c0_v,
             jnp.bool_(False), jnp.int32(0)))
        tau = (s_lo - ones_f) / c_lo

        if cp_out is not None:
            cp_out.wait()
            pprev_ref, pprev_n = prev

            def uzbody(e, _):
                pv = pprev_ref[pl.ds(e, L)]
                idx = pv[0] + lanes16
                plsc.store_scatter(out_v, [idx], zeros_f)
                return _

            lax.fori_loop(0, pprev_n, uzbody, jnp.int32(0))

        def sbody(e, _):
            pv = pl_ref[pl.ds(e, L)]
            idx = pv[0] + lanes16
            v = plsc.load_gather(in_ref, [idx])
            plsc.store_scatter(out_v, [idx], jnp.maximum(v - tau, zeros_f))
            return _

        lax.fori_loop(0, np_, sbody, jnp.int32(0))
        cp_out = pltpu.async_copy(out_v, out_hbm.at[row0 + i], sem_out)
        prev = (pl_ref, np_)

    cp_out.wait()


def kernel(input):
    mesh = plsc.VectorSubcoreMesh(core_axis_name="c", subcore_axis_name="s",
                                  num_cores=NC, num_subcores=NS)
    f = pl.kernel(
        _sparsemax_body,
        out_type=jax.ShapeDtypeStruct((R, N), jnp.float32),
        mesh=mesh,
        compiler_params=pltpu.CompilerParams(needs_layout_passes=False),
        scratch_types=[
            pltpu.VMEM((N,), jnp.float32),
            pltpu.VMEM((N,), jnp.float32),
            pltpu.VMEM((N,), jnp.float32),
            pltpu.VMEM((NG * L,), jnp.float32),
            pltpu.VMEM((NG * L + L,), jnp.int32),
            pltpu.VMEM((NG * L + L,), jnp.int32),
            pltpu.SemaphoreType.DMA,
            pltpu.SemaphoreType.DMA,
            pltpu.SemaphoreType.DMA,
        ],
    )
    return f(input)


# speedup vs baseline: 35.5281x; 1.1524x over previous
import jax
import jax.numpy as jnp
from jax import lax
from jax.experimental import pallas as pl
from jax.experimental.pallas import tpu as pltpu
from jax.experimental.pallas import tpu_sc as plsc

R, N = 128, 32768
L = 16
NC, NS = 2, 16
NW = NC * NS
RPW = R // NW
KPG = 16
GSZ = KPG * L
NG = N // GSZ
NEWTON_ITERS = 26


def _sparsemax_body(x_hbm, out_hbm, in0_v, in1_v, out_v, gmax_v,
                    pl0_v, pl1_v, sem_in0, sem_in1, sem_out):
    wid = lax.axis_index("c") * NS + lax.axis_index("s")
    lanes = lax.broadcasted_iota(jnp.int32, (L,), 0)
    lanes16 = lanes * L
    ones_i = jnp.ones((L,), jnp.int32)
    zeros_i = jnp.zeros((L,), jnp.int32)
    zeros_f = jnp.zeros((L,), jnp.float32)
    ones_f = jnp.ones((L,), jnp.float32)

    row0 = wid * RPW
    in_bufs = [in0_v, in1_v]
    in_sems = [sem_in0, sem_in1]
    pls = [pl0_v, pl1_v]

    cp_in = [None, None]
    cp_in[0] = pltpu.async_copy(x_hbm.at[row0], in0_v, sem_in0)

    def zbody(g, _):
        base = lax.shift_left(g, 8)
        for k in range(KPG):
            out_v[pl.ds(base + k * L, L)] = zeros_f
        return _

    lax.fori_loop(0, NG, zbody, jnp.int32(0))

    cp_out = None
    prev = None

    for i in range(RPW):
        b = i & 1
        in_ref = in_bufs[b]
        pl_ref = pls[b]
        cp_in[b].wait()
        if i + 1 < RPW:
            cp_in[1 - b] = pltpu.async_copy(
                x_hbm.at[row0 + i + 1], in_bufs[1 - b], in_sems[1 - b])

        def maxbody(g, acc):
            base = lax.shift_left(g, 8)
            gm = in_ref[pl.ds(base, L)]
            for k in range(1, KPG):
                gm = jnp.maximum(gm, in_ref[pl.ds(base + k * L, L)])
            gmax_v[pl.ds(lax.shift_left(g, 4), L)] = gm
            return jnp.maximum(acc, gm)

        acc = lax.fori_loop(0, NG, maxbody,
                            jnp.full((L,), -jnp.inf, jnp.float32))
        rowmax = jnp.max(acc)
        tau0 = rowmax - 1.0

        def gscan(g, np_):
            gm = gmax_v[pl.ds(lax.shift_left(g, 4), L)]
            m = gm > tau0

            def compact(o):
                plsc.store_compressed(pl_ref.at[pl.ds(o, L)],
                                      lax.shift_left(g, 8) + lanes, mask=m)
                return o + jnp.sum(jnp.where(m, ones_i, zeros_i))

            return lax.cond(jnp.any(m), compact, lambda o: o, np_)

        np_ = lax.fori_loop(0, NG, gscan, jnp.int32(0))

        def evalf(t_v):
            nq = lax.shift_right_logical(np_ + 3, 2)

            def ebody(q, carry):
                sv, cv = carry
                e = lax.shift_left(q, 2)
                pv = pl_ref[pl.ds(e, L)]
                for k in range(4):
                    ok = (e + k) < np_
                    okv = jnp.broadcast_to(jnp.where(ok, 1, 0), (L,))
                    p = jnp.where(ok, pv[k], pv[0])
                    v = plsc.load_gather(in_ref, [p + lanes16])
                    m = jnp.logical_and(v > t_v, okv > zeros_i)
                    sv = sv + jnp.where(m, v, zeros_f)
                    cv = cv + jnp.where(m, ones_f, zeros_f)
                return sv, cv

            sv, cv = lax.fori_loop(0, nq, ebody, (zeros_f, zeros_f))
            return (jnp.broadcast_to(jnp.sum(sv), (L,)),
                    jnp.broadcast_to(jnp.sum(cv), (L,)))

        tau0_v = jnp.broadcast_to(tau0, (L,))
        s0_v, c0_v = evalf(tau0_v)

        def ncond(carry):
            _, _, _, _, done, it = carry
            return jnp.logical_and(jnp.logical_not(done), it < NEWTON_ITERS)

        def nbody(carry):
            lo, hi, s_lo, c_lo, _, it = carry
            t_newton = (s_lo - ones_f) / c_lo
            fixedpt = jnp.all(t_newton <= lo)
            t = jnp.minimum(jnp.maximum(t_newton, 0.5 * (lo + hi)), hi)
            s_t, c_t = evalf(t)
            f_t = s_t - t * c_t - ones_f
            good = f_t >= zeros_f
            stable = jnp.all(
                jnp.logical_and(jnp.logical_and(good, c_t == c_lo),
                                t == t_newton))
            done = jnp.logical_or(fixedpt, stable)
            return (jnp.where(good, t, lo), jnp.where(good, hi, t),
                    jnp.where(good, s_t, s_lo), jnp.where(good, c_t, c_lo),
                    done, it + 1)

        lo, hi, s_lo, c_lo, _, _ = lax.while_loop(
            ncond, nbody,
            (tau0_v, jnp.broadcast_to(rowmax, (L,)), s0_v, c0_v,
             jnp.bool_(False), jnp.int32(0)))
        tau = (s_lo - ones_f) / c_lo

        if cp_out is not None:
            cp_out.wait()
            pprev_ref, pprev_n = prev

            def uzbody(e, _):
                pv = pprev_ref[pl.ds(e, L)]
                idx = pv[0] + lanes16
                plsc.store_scatter(out_v, [idx], zeros_f)
                return _

            lax.fori_loop(0, pprev_n, uzbody, jnp.int32(0))

        def sbody(e, _):
            pv = pl_ref[pl.ds(e, L)]
            idx = pv[0] + lanes16
            v = plsc.load_gather(in_ref, [idx])
            plsc.store_scatter(out_v, [idx], jnp.maximum(v - tau, zeros_f))
            return _

        lax.fori_loop(0, np_, sbody, jnp.int32(0))
        cp_out = pltpu.async_copy(out_v, out_hbm.at[row0 + i], sem_out)
        prev = (pl_ref, np_)

    cp_out.wait()


def kernel(input):
    mesh = plsc.VectorSubcoreMesh(core_axis_name="c", subcore_axis_name="s",
                                  num_cores=NC, num_subcores=NS)
    f = pl.kernel(
        _sparsemax_body,
        out_type=jax.ShapeDtypeStruct((R, N), jnp.float32),
        mesh=mesh,
        compiler_params=pltpu.CompilerParams(needs_layout_passes=False),
        scratch_types=[
            pltpu.VMEM((N,), jnp.float32),
            pltpu.VMEM((N,), jnp.float32),
            pltpu.VMEM((N,), jnp.float32),
            pltpu.VMEM((NG * L,), jnp.float32),
            pltpu.VMEM((NG * L + L,), jnp.int32),
            pltpu.VMEM((NG * L + L,), jnp.int32),
            pltpu.SemaphoreType.DMA,
            pltpu.SemaphoreType.DMA,
            pltpu.SemaphoreType.DMA,
        ],
    )
    return f(input)


# speedup vs baseline: 38.5921x; 1.0862x over previous
import jax
import jax.numpy as jnp
from jax import lax
from jax.experimental import pallas as pl
from jax.experimental.pallas import tpu as pltpu
from jax.experimental.pallas import tpu_sc as plsc

R, N = 128, 32768
L = 16
NC, NS = 2, 16
NW = NC * NS
RPW = R // NW
KPG = 16
GSZ = KPG * L
NG = N // GSZ
NEWTON_ITERS = 26
TC_BLK = 16


def _tau_body(x_hbm, tau_hbm, in0_v, in1_v, gmax_v, pl_v, tau_v,
              sem_in0, sem_in1, sem_tau):
    wid = lax.axis_index("c") * NS + lax.axis_index("s")
    lanes = lax.broadcasted_iota(jnp.int32, (L,), 0)
    ones_i = jnp.ones((L,), jnp.int32)
    zeros_i = jnp.zeros((L,), jnp.int32)
    zeros_f = jnp.zeros((L,), jnp.float32)
    ones_f = jnp.ones((L,), jnp.float32)

    row0 = wid * RPW
    in_bufs = [in0_v, in1_v]
    in_sems = [sem_in0, sem_in1]

    cp_in = [None, None]
    cp_in[0] = pltpu.async_copy(x_hbm.at[row0], in0_v, sem_in0)

    cp_tau = None

    for i in range(RPW):
        b = i & 1
        in_ref = in_bufs[b]
        cp_in[b].wait()
        if i + 1 < RPW:
            cp_in[1 - b] = pltpu.async_copy(
                x_hbm.at[row0 + i + 1], in_bufs[1 - b], in_sems[1 - b])

        def maxbody(g, acc):
            base = lax.shift_left(g, 8)
            gm = in_ref[pl.ds(base, L)]
            for k in range(1, KPG):
                gm = jnp.maximum(gm, in_ref[pl.ds(base + k * L, L)])
            gmax_v[pl.ds(lax.shift_left(g, 4), L)] = gm
            return jnp.maximum(acc, gm)

        acc = lax.fori_loop(0, NG, maxbody,
                            jnp.full((L,), -jnp.inf, jnp.float32))
        rowmax = jnp.max(acc)
        tau0 = rowmax - 1.0

        tau0_v = jnp.broadcast_to(tau0, (L,))

        def gscan(g, npvm1):
            gm = gmax_v[pl.ds(lax.shift_left(g, 4), L)]
            m = gm > tau0_v
            r = plsc.cumsum(jnp.where(m, ones_i, zeros_i))
            ids = lax.shift_left(g, 8) + lanes
            plsc.store_scatter(pl_v, [npvm1 + r], ids, mask=m)
            return npvm1 + plsc.all_reduce_population_count(m)

        npvm1 = lax.fori_loop(0, NG, gscan,
                              jnp.full((L,), -1, jnp.int32))
        np_ = npvm1[0] + 1

        npv = npvm1 + ones_i
        nq = lax.shift_right_logical(np_ + (L - 1), 4)

        def evalf(t_v):
            def ebody(q, carry):
                sv, cv = carry
                base = lax.shift_left(q, 4)
                pv = pl_v[pl.ds(base, L)]
                em = (base + lanes) < npv
                pvs = jnp.where(em, pv, zeros_i)
                for j in range(KPG):
                    v = plsc.load_gather(in_ref, [pvs + (j * L)])
                    m = jnp.logical_and(v > t_v, em)
                    sv = sv + jnp.where(m, v, zeros_f)
                    cv = cv + jnp.where(m, ones_f, zeros_f)
                return sv, cv

            sv, cv = lax.fori_loop(0, nq, ebody, (zeros_f, zeros_f))
            return (jnp.broadcast_to(jnp.sum(sv), (L,)),
                    jnp.broadcast_to(jnp.sum(cv), (L,)))

        s0_v, c0_v = evalf(tau0_v)

        def ncond(carry):
            _, _, _, _, done, it = carry
            return jnp.logical_and(jnp.logical_not(done), it < NEWTON_ITERS)

        def nbody(carry):
            lo, hi, s_lo, c_lo, _, it = carry
            t_newton = (s_lo - ones_f) / c_lo
            fixedpt = jnp.all(t_newton <= lo)
            t = jnp.minimum(jnp.maximum(t_newton, 0.5 * (lo + hi)), hi)
            s_t, c_t = evalf(t)
            f_t = s_t - t * c_t - ones_f
            good = f_t >= zeros_f
            stable = jnp.all(
                jnp.logical_and(jnp.logical_and(good, c_t == c_lo),
                                t == t_newton))
            done = jnp.logical_or(fixedpt, stable)
            return (jnp.where(good, t, lo), jnp.where(good, hi, t),
                    jnp.where(good, s_t, s_lo), jnp.where(good, c_t, c_lo),
                    done, it + 1)

        lo, hi, s_lo, c_lo, _, _ = lax.while_loop(
            ncond, nbody,
            (tau0_v, jnp.broadcast_to(rowmax, (L,)), s0_v, c0_v,
             jnp.bool_(False), jnp.int32(0)))
        tau = (s_lo - ones_f) / c_lo

        if cp_tau is not None:
            cp_tau.wait()
        tau_v[pl.ds(0, L)] = tau
        cp_tau = pltpu.async_copy(tau_v, tau_hbm.at[row0 + i], sem_tau)

    cp_tau.wait()


def _relu_body(tau_ref, x_ref, o_ref):
    o_ref[...] = jnp.maximum(x_ref[...] - tau_ref[:, :1], 0.0)


def kernel(input):
    mesh = plsc.VectorSubcoreMesh(core_axis_name="c", subcore_axis_name="s",
                                  num_cores=NC, num_subcores=NS)
    tau_f = pl.kernel(
        _tau_body,
        out_type=jax.ShapeDtypeStruct((R, L), jnp.float32),
        mesh=mesh,
        compiler_params=pltpu.CompilerParams(needs_layout_passes=False),
        scratch_types=[
            pltpu.VMEM((N,), jnp.float32),
            pltpu.VMEM((N,), jnp.float32),
            pltpu.VMEM((NG * L,), jnp.float32),
            pltpu.VMEM((NG * L + L,), jnp.int32),
            pltpu.VMEM((L,), jnp.float32),
            pltpu.SemaphoreType.DMA,
            pltpu.SemaphoreType.DMA,
            pltpu.SemaphoreType.DMA,
        ],
    )
    taus = tau_f(input)

    return pl.pallas_call(
        _relu_body,
        grid=(R // TC_BLK,),
        in_specs=[
            pl.BlockSpec((TC_BLK, L), lambda i: (i, 0)),
            pl.BlockSpec((TC_BLK, N), lambda i: (i, 0)),
        ],
        out_specs=pl.BlockSpec((TC_BLK, N), lambda i: (i, 0)),
        out_shape=jax.ShapeDtypeStruct((R, N), jnp.float32),
    )(taus, input)


# speedup vs baseline: 44.3966x; 1.1504x over previous
import jax
import jax.numpy as jnp
from jax import lax
from jax.experimental import pallas as pl
from jax.experimental.pallas import tpu as pltpu
from jax.experimental.pallas import tpu_sc as plsc

R, N = 128, 32768
L = 16
NC, NS = 2, 16
NW = NC * NS
RPW = R // NW
KPG = 16
GSZ = KPG * L
NG = N // GSZ
NEWTON_ITERS = 26


def _sparsemax_body(x_hbm, out_hbm, in0_v, in1_v, out_v, gmax_v,
                    pl0_v, pl1_v, sem_in0, sem_in1, sem_out):
    wid = lax.axis_index("c") * NS + lax.axis_index("s")
    lanes = lax.broadcasted_iota(jnp.int32, (L,), 0)
    lanes16 = lanes * L
    ones_i = jnp.ones((L,), jnp.int32)
    zeros_i = jnp.zeros((L,), jnp.int32)
    zeros_f = jnp.zeros((L,), jnp.float32)
    ones_f = jnp.ones((L,), jnp.float32)

    row0 = wid * RPW
    in_bufs = [in0_v, in1_v]
    in_sems = [sem_in0, sem_in1]
    pls = [pl0_v, pl1_v]

    cp_in = [None, None]
    cp_in[0] = pltpu.async_copy(x_hbm.at[row0], in0_v, sem_in0)

    def zbody(g, _):
        base = lax.shift_left(g, 8)
        for k in range(KPG):
            out_v[pl.ds(base + k * L, L)] = zeros_f
        return _

    lax.fori_loop(0, NG, zbody, jnp.int32(0))

    cp_out = None
    prev = None

    for i in range(RPW):
        b = i & 1
        in_ref = in_bufs[b]
        pl_ref = pls[b]
        cp_in[b].wait()
        if i + 1 < RPW:
            cp_in[1 - b] = pltpu.async_copy(
                x_hbm.at[row0 + i + 1], in_bufs[1 - b], in_sems[1 - b])

        def maxbody(g, acc):
            base = lax.shift_left(g, 8)
            a = [in_ref[pl.ds(base + k * L, L)] for k in range(4)]
            for k in range(4, KPG):
                a[k & 3] = jnp.maximum(a[k & 3],
                                       in_ref[pl.ds(base + k * L, L)])
            gm = jnp.maximum(jnp.maximum(a[0], a[1]),
                             jnp.maximum(a[2], a[3]))
            gmax_v[pl.ds(lax.shift_left(g, 4), L)] = gm
            return jnp.maximum(acc, gm)

        acc = lax.fori_loop(0, NG, maxbody,
                            jnp.full((L,), -jnp.inf, jnp.float32))
        rowmax = jnp.max(acc)
        tau0 = rowmax - 1.0

        tau0_v = jnp.broadcast_to(tau0, (L,))

        def gscan(g, npvm1):
            gm = gmax_v[pl.ds(lax.shift_left(g, 4), L)]
            m = gm > tau0_v
            r = plsc.cumsum(jnp.where(m, ones_i, zeros_i))
            ids = lax.shift_left(g, 8) + lanes
            plsc.store_scatter(pl_ref, [npvm1 + r], ids, mask=m)
            return npvm1 + plsc.all_reduce_population_count(m)

        npvm1 = lax.fori_loop(0, NG, gscan,
                              jnp.full((L,), -1, jnp.int32))
        np_ = npvm1[0] + 1

        npv = npvm1 + ones_i
        nq = lax.shift_right_logical(np_ + (L - 1), 4)

        def evalf(t_v):
            def ebody(q, carry):
                sv, cv = carry
                base = lax.shift_left(q, 4)
                pv = pl_ref[pl.ds(base, L)]
                em = (base + lanes) < npv
                pvs = jnp.where(em, pv, zeros_i)
                for j in range(KPG):
                    v = plsc.load_gather(in_ref, [pvs + (j * L)])
                    m = jnp.logical_and(v > t_v, em)
                    sv = sv + jnp.where(m, v, zeros_f)
                    cv = cv + jnp.where(m, ones_f, zeros_f)
                return sv, cv

            sv, cv = lax.fori_loop(0, nq, ebody, (zeros_f, zeros_f))
            return (jnp.broadcast_to(jnp.sum(sv), (L,)),
                    jnp.broadcast_to(jnp.sum(cv), (L,)))

        s0_v, c0_v = evalf(tau0_v)

        def ncond(carry):
            _, _, _, _, done, it = carry
            return jnp.logical_and(jnp.logical_not(done), it < NEWTON_ITERS)

        def nbody(carry):
            lo, hi, s_lo, c_lo, _, it = carry
            t_newton = (s_lo - ones_f) / c_lo
            fixedpt = jnp.all(t_newton <= lo)
            t = jnp.minimum(jnp.maximum(t_newton, 0.5 * (lo + hi)), hi)
            s_t, c_t = evalf(t)
            f_t = s_t - t * c_t - ones_f
            good = f_t >= zeros_f
            stable = jnp.all(
                jnp.logical_and(jnp.logical_and(good, c_t == c_lo),
                                t == t_newton))
            done = jnp.logical_or(fixedpt, stable)
            return (jnp.where(good, t, lo), jnp.where(good, hi, t),
                    jnp.where(good, s_t, s_lo), jnp.where(good, c_t, c_lo),
                    done, it + 1)

        lo, hi, s_lo, c_lo, _, _ = lax.while_loop(
            ncond, nbody,
            (tau0_v, jnp.broadcast_to(rowmax, (L,)), s0_v, c0_v,
             jnp.bool_(False), jnp.int32(0)))
        tau = (s_lo - ones_f) / c_lo

        if cp_out is not None:
            cp_out.wait()

            pprev_ref, npv_prev, nq_prev = prev

            def uzbody(q, _):
                base = lax.shift_left(q, 4)
                pv = pprev_ref[pl.ds(base, L)]
                em = (base + lanes) < npv_prev
                pvs = jnp.where(em, pv, zeros_i)
                for j in range(KPG):
                    plsc.store_scatter(out_v, [pvs + (j * L)], zeros_f,
                                       mask=em)
                return _

            lax.fori_loop(0, nq_prev, uzbody, jnp.int32(0))

        def sbody(q, _):
            base = lax.shift_left(q, 4)
            pv = pl_ref[pl.ds(base, L)]
            em = (base + lanes) < npv
            pvs = jnp.where(em, pv, zeros_i)
            for j in range(KPG):
                idx = pvs + (j * L)
                v = plsc.load_gather(in_ref, [idx])
                plsc.store_scatter(out_v, [idx],
                                   jnp.maximum(v - tau, zeros_f), mask=em)
            return _

        lax.fori_loop(0, nq, sbody, jnp.int32(0))
        cp_out = pltpu.async_copy(out_v, out_hbm.at[row0 + i], sem_out)
        prev = (pl_ref, npv, nq)

    cp_out.wait()


def kernel(input):
    mesh = plsc.VectorSubcoreMesh(core_axis_name="c", subcore_axis_name="s",
                                  num_cores=NC, num_subcores=NS)
    f = pl.kernel(
        _sparsemax_body,
        out_type=jax.ShapeDtypeStruct((R, N), jnp.float32),
        mesh=mesh,
        compiler_params=pltpu.CompilerParams(needs_layout_passes=False),
        scratch_types=[
            pltpu.VMEM((N,), jnp.float32),
            pltpu.VMEM((N,), jnp.float32),
            pltpu.VMEM((N,), jnp.float32),
            pltpu.VMEM((NG * L,), jnp.float32),
            pltpu.VMEM((NG * L + L,), jnp.int32),
            pltpu.VMEM((NG * L + L,), jnp.int32),
            pltpu.SemaphoreType.DMA,
            pltpu.SemaphoreType.DMA,
            pltpu.SemaphoreType.DMA,
        ],
    )
    return f(input)


# speedup vs baseline: 44.8794x; 1.0109x over previous
import jax
import jax.numpy as jnp
from jax import lax
from jax.experimental import pallas as pl
from jax.experimental.pallas import tpu as pltpu
from jax.experimental.pallas import tpu_sc as plsc

R, N = 128, 32768
L = 16
NC, NS = 2, 16
NW = NC * NS
RPW = R // NW
KPG = 16
GSZ = KPG * L
NG = N // GSZ
NEWTON_ITERS = 26


def _sparsemax_body(x_hbm, out_hbm, in0_v, in1_v, out_v, gmax_v,
                    pl0_v, pl1_v, sem_in0, sem_in1, sem_out):
    wid = lax.axis_index("c") * NS + lax.axis_index("s")
    lanes = lax.broadcasted_iota(jnp.int32, (L,), 0)
    lanes16 = lanes * L
    ones_i = jnp.ones((L,), jnp.int32)
    zeros_i = jnp.zeros((L,), jnp.int32)
    zeros_f = jnp.zeros((L,), jnp.float32)
    ones_f = jnp.ones((L,), jnp.float32)

    row0 = wid * RPW
    in_bufs = [in0_v, in1_v]
    in_sems = [sem_in0, sem_in1]
    pls = [pl0_v, pl1_v]

    cp_in = [None, None]
    cp_in[0] = pltpu.async_copy(x_hbm.at[row0], in0_v, sem_in0)

    def zbody(g, _):
        base = lax.shift_left(g, 8)
        for k in range(KPG):
            out_v[pl.ds(base + k * L, L)] = zeros_f
        return _

    lax.fori_loop(0, NG, zbody, jnp.int32(0))

    cp_out = None
    prev = None

    for i in range(RPW):
        b = i & 1
        in_ref = in_bufs[b]
        pl_ref = pls[b]
        cp_in[b].wait()
        if i + 1 < RPW:
            cp_in[1 - b] = pltpu.async_copy(
                x_hbm.at[row0 + i + 1], in_bufs[1 - b], in_sems[1 - b])

        def maxbody(g, acc):
            base = lax.shift_left(g, 8)
            gm = in_ref[pl.ds(base, L)]
            for k in range(1, KPG):
                gm = jnp.maximum(gm, in_ref[pl.ds(base + k * L, L)])
            gmax_v[pl.ds(lax.shift_left(g, 4), L)] = gm
            return jnp.maximum(acc, gm)

        acc = lax.fori_loop(0, NG, maxbody,
                            jnp.full((L,), -jnp.inf, jnp.float32))
        rowmax = jnp.max(acc)
        tau0 = rowmax - 1.0

        tau0_v = jnp.broadcast_to(tau0, (L,))

        def gscan(g, npvm1):
            gm = gmax_v[pl.ds(lax.shift_left(g, 4), L)]
            m = gm > tau0_v
            r = plsc.cumsum(jnp.where(m, ones_i, zeros_i))
            ids = lax.shift_left(g, 8) + lanes
            plsc.store_scatter(pl_ref, [npvm1 + r], ids, mask=m)
            return npvm1 + plsc.all_reduce_population_count(m)

        npvm1 = lax.fori_loop(0, NG, gscan,
                              jnp.full((L,), -1, jnp.int32))
        np_ = npvm1[0] + 1

        npv = npvm1 + ones_i
        nq = lax.shift_right_logical(np_ + (L - 1), 4)

        def evalf(t_v):
            def ebody(q, carry):
                sv, cv = carry
                base = lax.shift_left(q, 4)
                pv = pl_ref[pl.ds(base, L)]
                em = (base + lanes) < npv
                pvs = jnp.where(em, pv, zeros_i)
                for j in range(KPG):
                    v = plsc.load_gather(in_ref, [pvs + (j * L)])
                    m = jnp.logical_and(v > t_v, em)
                    sv = sv + jnp.where(m, v, zeros_f)
                    cv = cv + jnp.where(m, ones_f, zeros_f)
                return sv, cv

            sv, cv = lax.fori_loop(0, nq, ebody, (zeros_f, zeros_f))
            return (jnp.broadcast_to(jnp.sum(sv), (L,)),
                    jnp.broadcast_to(jnp.sum(cv), (L,)))

        s0_v, c0_v = evalf(tau0_v)

        def ncond(carry):
            _, _, _, _, done, it = carry
            return jnp.logical_and(jnp.logical_not(done), it < NEWTON_ITERS)

        def nbody(carry):
            lo, hi, s_lo, c_lo, _, it = carry
            t_newton = (s_lo - ones_f) / c_lo
            fixedpt = jnp.all(t_newton <= lo)
            t = jnp.minimum(jnp.maximum(t_newton, 0.5 * (lo + hi)), hi)
            s_t, c_t = evalf(t)
            f_t = s_t - t * c_t - ones_f
            good = f_t >= zeros_f
            stable = jnp.all(
                jnp.logical_and(jnp.logical_and(good, c_t == c_lo),
                                t == t_newton))
            done = jnp.logical_or(fixedpt, stable)
            return (jnp.where(good, t, lo), jnp.where(good, hi, t),
                    jnp.where(good, s_t, s_lo), jnp.where(good, c_t, c_lo),
                    done, it + 1)

        lo, hi, s_lo, c_lo, _, _ = lax.while_loop(
            ncond, nbody,
            (tau0_v, jnp.broadcast_to(rowmax, (L,)), s0_v, c0_v,
             jnp.bool_(False), jnp.int32(0)))
        tau = (s_lo - ones_f) / c_lo

        if cp_out is not None:
            cp_out.wait()

            pprev_ref, npv_prev, nq_prev = prev

            def uzbody(q, _):
                base = lax.shift_left(q, 4)
                pv = pprev_ref[pl.ds(base, L)]
                em = (base + lanes) < npv_prev
                pvs = jnp.where(em, pv, zeros_i)
                for j in range(KPG):
                    plsc.store_scatter(out_v, [pvs + (j * L)], zeros_f,
                                       mask=em)
                return _

            lax.fori_loop(0, nq_prev, uzbody, jnp.int32(0))

        def sbody(q, _):
            base = lax.shift_left(q, 4)
            pv = pl_ref[pl.ds(base, L)]
            em = (base + lanes) < npv
            pvs = jnp.where(em, pv, zeros_i)
            for j in range(KPG):
                idx = pvs + (j * L)
                v = plsc.load_gather(in_ref, [idx])
                plsc.store_scatter(out_v, [idx],
                                   jnp.maximum(v - tau, zeros_f), mask=em)
            return _

        lax.fori_loop(0, nq, sbody, jnp.int32(0))
        cp_out = pltpu.async_copy(out_v, out_hbm.at[row0 + i], sem_out)
        prev = (pl_ref, npv, nq)

    cp_out.wait()


def kernel(input):
    mesh = plsc.VectorSubcoreMesh(core_axis_name="c", subcore_axis_name="s",
                                  num_cores=NC, num_subcores=NS)
    f = pl.kernel(
        _sparsemax_body,
        out_type=jax.ShapeDtypeStruct((R, N), jnp.float32),
        mesh=mesh,
        compiler_params=pltpu.CompilerParams(needs_layout_passes=False),
        scratch_types=[
            pltpu.VMEM((N,), jnp.float32),
            pltpu.VMEM((N,), jnp.float32),
            pltpu.VMEM((N,), jnp.float32),
            pltpu.VMEM((NG * L,), jnp.float32),
            pltpu.VMEM((NG * L + L,), jnp.int32),
            pltpu.VMEM((NG * L + L,), jnp.int32),
            pltpu.SemaphoreType.DMA,
            pltpu.SemaphoreType.DMA,
            pltpu.SemaphoreType.DMA,
        ],
    )
    return f(input)
